# Initial kernel scaffold; baseline (speedup 1.0000x reference)
import jax
import jax.numpy as jnp
from jax.experimental import pallas as pl


def kernel(xc, yc, xt, pos, senders, receivers, params):
    raise NotImplementedError("write your pallas kernel here")


# speedup vs baseline: 3.7438x; 3.7438x over previous
import functools

import jax
import jax.numpy as jnp
from jax import lax
from jax.experimental import pallas as pl
from jax.experimental.pallas import tpu as pltpu
from jax.experimental.pallas import tpu_sc as plsc

B, NC, NT, N, E = 4, 1024, 1024, 2048, 16384
DX, DH, NHEAD = 2, 128, 8
DN = DH + DX
F = 144
STEPS = 3
EPS = 1e-5
BN = B * N
BE = B * E

NWORK = 32
EPW = E // NWORK
CH = 128
NCHUNK = EPW // CH


def _fullspec(shape):
    nd = len(shape)
    return pl.BlockSpec(shape, lambda b, _n=nd: (0,) * _n)


def _bspec(shape):
    nd = len(shape)
    return pl.BlockSpec(shape, lambda b, _n=nd: (b,) + (0,) * (_n - 1))


def _enc_body(xc_ref, yc_ref, posT_ref, w1x_ref, w1y_ref, b1_ref, w2_ref,
              b2_ref, w3_ref, b3_ref, h0_ref):
    xcb = xc_ref[0]
    ycb = yc_ref[0]
    posT = posT_ref[...]
    x = jnp.dot(xcb, w1x_ref[...], preferred_element_type=jnp.float32, precision=lax.Precision.HIGHEST)
    x = x + ycb * w1y_ref[...] + b1_ref[...]
    x = jnp.maximum(x, 0.0)
    x = jnp.maximum(jnp.dot(x, w2_ref[...], preferred_element_type=jnp.float32, precision=lax.Precision.HIGHEST)
                    + b2_ref[...], 0.0)
    emb = jnp.dot(x, w3_ref[...], preferred_element_type=jnp.float32, precision=lax.Precision.HIGHEST) + b3_ref[...]

    cross = jnp.dot(xcb, posT, preferred_element_type=jnp.float32, precision=lax.Precision.HIGHEST)
    xx = jnp.sum(xcb * xcb, axis=1, keepdims=True)
    pp = jnp.sum(posT * posT, axis=0, keepdims=True)
    logits = 2.0 * cross - xx - pp
    logits = logits - jnp.max(logits, axis=1, keepdims=True)
    ex = jnp.exp(logits)
    scores = ex / jnp.sum(ex, axis=1, keepdims=True)
    h0 = lax.dot_general(scores, emb, (((0,), (0,)), ((), ())),
                         preferred_element_type=jnp.float32, precision=lax.Precision.HIGHEST)
    h0_ref[0] = h0


def _encoder(xc, yc, posT, w1x, w1y, b1, w2, b2, w3, b3):
    return pl.pallas_call(
        _enc_body,
        grid=(B,),
        in_specs=[
            _bspec((1, NC, DX)), _bspec((1, NC, 1)), _fullspec((DX, N)),
            _fullspec((DX, DH)), _fullspec((1, DH)), _fullspec((1, DH)),
            _fullspec((DH, DH)), _fullspec((1, DH)),
            _fullspec((DH, DH)), _fullspec((1, DH)),
        ],
        out_specs=_bspec((1, N, DH)),
        out_shape=jax.ShapeDtypeStruct((B, N, DH), jnp.float32),
    )(xc, yc, posT, w1x, w1y, b1, w2, b2, w3, b3)


def _proj_body(h_ref, pos_ref, axy_ref, ahh_ref, bm_ref, cxy_ref, chh_ref,
               p_ref, s_ref):
    hb = h_ref[0]
    posb = pos_ref[...]
    p_ref[0] = (jnp.dot(posb, axy_ref[...], preferred_element_type=jnp.float32, precision=lax.Precision.HIGHEST)
                + jnp.dot(hb, ahh_ref[...], preferred_element_type=jnp.float32, precision=lax.Precision.HIGHEST)
                + bm_ref[...])
    s_ref[0] = (jnp.dot(posb, cxy_ref[...], preferred_element_type=jnp.float32, precision=lax.Precision.HIGHEST)
                + jnp.dot(hb, chh_ref[...], preferred_element_type=jnp.float32, precision=lax.Precision.HIGHEST))


def _proj(h, pos, axy, ahh, bm, cxy, chh):
    return pl.pallas_call(
        _proj_body,
        grid=(B,),
        in_specs=[
            _bspec((1, N, DH)), _fullspec((N, DX)),
            _fullspec((DX, F)), _fullspec((DH, F)), _fullspec((1, F)),
            _fullspec((DX, F)), _fullspec((DH, F)),
        ],
        out_specs=[_bspec((1, N, F)), _bspec((1, N, F))],
        out_shape=[jax.ShapeDtypeStruct((B, N, F), jnp.float32),
                   jax.ShapeDtypeStruct((B, N, F), jnp.float32)],
    )(h, pos, axy, ahh, bm, cxy, chh)


_LNROWS = 4096


def _msgln_body(g_ref, g1_ref, b1_ref, o_ref):
    x = g_ref[...]
    s = jnp.sum(x, axis=1, keepdims=True) * (1.0 / DN)
    q = jnp.sum(x * x, axis=1, keepdims=True) * (1.0 / DN)
    var = jnp.maximum(q - s * s, 0.0)
    rs = lax.rsqrt(var + EPS)
    o_ref[...] = (x - s) * rs * g1_ref[...] + b1_ref[...]


def _msgln(g, g1p, b1p):
    nb = BE // _LNROWS
    return pl.pallas_call(
        _msgln_body,
        grid=(nb,),
        in_specs=[_bspec((_LNROWS, F)), _fullspec((1, F)), _fullspec((1, F))],
        out_specs=_bspec((_LNROWS, F)),
        out_shape=jax.ShapeDtypeStruct((BE, F), jnp.float32),
    )(g, g1p, b1p)


def _sc_gather_body(p_hbm, s_hbm, recv_hbm, send_hbm, g_hbm,
                    idxr, idxs, prow, srow, gbuf, sem):
    cid = lax.axis_index("c")
    sid = lax.axis_index("s")
    wid = cid * 16 + sid

    def chunk(t, carry):
        b = t // (NCHUNK)
        k = t - b * NCHUNK
        off = b * N
        ebase = pl.multiple_of(wid * EPW + k * CH, CH)
        pltpu.sync_copy(recv_hbm.at[pl.ds(ebase, CH)], idxr)
        pltpu.sync_copy(send_hbm.at[pl.ds(ebase, CH)], idxs)
        for i in range(CH // 16):
            sl = pl.ds(i * 16, 16)
            idxr[sl] = idxr[sl] + off
            idxs[sl] = idxs[sl] + off
        pltpu.async_copy(p_hbm.at[idxr], prow, sem).wait()
        pltpu.async_copy(s_hbm.at[idxs], srow, sem).wait()

        def addb(e, c2):
            for j in range(F // 16):
                sl = pl.ds(j * 16, 16)
                gbuf[e, sl] = prow[e, sl] + srow[e, sl]
            return c2

        lax.fori_loop(0, CH, addb, 0)
        gb = pl.multiple_of(b * E + ebase, CH)
        pltpu.sync_copy(gbuf, g_hbm.at[pl.ds(gb, CH)])
        return carry

    lax.fori_loop(0, B * NCHUNK, chunk, 0)


def _sc_gather(p2, s2, receivers, senders):
    call = pl.kernel(
        _sc_gather_body,
        out_type=jax.ShapeDtypeStruct((BE, F), jnp.float32),
        compiler_params=pltpu.CompilerParams(use_tc_tiling_on_sc=False),
        mesh=plsc.VectorSubcoreMesh(core_axis_name="c", subcore_axis_name="s"),
        scratch_types=[
            pltpu.VMEM((CH,), jnp.int32),
            pltpu.VMEM((CH,), jnp.int32),
            pltpu.VMEM((CH, F), jnp.float32),
            pltpu.VMEM((CH, F), jnp.float32),
            pltpu.VMEM((CH, F), jnp.float32),
            pltpu.SemaphoreType.DMA,
        ],
    )
    return call(p2, s2, receivers, senders)


def _sc_scatter_body(m_hbm, recv_hbm, inb_hbm, idxr, mbuf, spmem, sem):
    cid = lax.axis_index("c")
    sid = lax.axis_index("s")

    def zb(e, c2):
        for j in range(F // 16):
            mbuf[e, pl.ds(j * 16, 16)] = jnp.zeros((16,), jnp.float32)
        return c2

    lax.fori_loop(0, CH, zb, 0)

    def zcp(k, c2):
        r0 = pl.multiple_of(sid * (BN // 16) + k * CH, CH)
        pltpu.sync_copy(mbuf, spmem.at[pl.ds(r0, CH)])
        return c2

    lax.fori_loop(0, BN // 16 // CH, zcp, 0)
    plsc.subcore_barrier()

    def sb(t, c2):
        b = t // NCHUNK
        k = t - b * NCHUNK
        ebase = pl.multiple_of(cid * (E // 2) + sid * EPW + k * CH, CH)
        pltpu.sync_copy(recv_hbm.at[pl.ds(ebase, CH)], idxr)
        for i in range(CH // 16):
            sl = pl.ds(i * 16, 16)
            idxr[sl] = idxr[sl] + b * N
        mb = pl.multiple_of(b * E + ebase, CH)
        pltpu.sync_copy(m_hbm.at[pl.ds(mb, CH)], mbuf)
        pltpu.sync_copy(mbuf, spmem.at[idxr], add=True)
        return c2

    lax.fori_loop(0, B * NCHUNK, sb, 0)
    plsc.subcore_barrier()

    def db(k, c2):
        r0 = pl.multiple_of(sid * (BN // 16) + k * CH, CH)
        pltpu.sync_copy(spmem.at[pl.ds(r0, CH)], mbuf)
        pltpu.sync_copy(mbuf, inb_hbm.at[cid, pl.ds(r0, CH)])
        return c2

    lax.fori_loop(0, BN // 16 // CH, db, 0)


def _sc_scatter(msgs, receivers):
    call = pl.kernel(
        _sc_scatter_body,
        out_type=jax.ShapeDtypeStruct((2, BN, F), jnp.float32),
        compiler_params=pltpu.CompilerParams(use_tc_tiling_on_sc=False),
        mesh=plsc.VectorSubcoreMesh(core_axis_name="c", subcore_axis_name="s"),
        scratch_types=[
            pltpu.VMEM((CH,), jnp.int32),
            pltpu.VMEM((CH, F), jnp.float32),
            pltpu.VMEM_SHARED((BN, F), jnp.float32),
            pltpu.SemaphoreType.DMA,
        ],
    )
    return call(msgs, receivers)


def _upd_body(h_ref, pos_ref, inb_ref, wnxy_ref, wnhh_ref, wn2_ref, bn_ref,
              g2_ref, b2_ref, o_ref):
    hb = h_ref[0]
    posb = pos_ref[...]
    ib = inb_ref[0, 0] + inb_ref[1, 0]
    u = (jnp.dot(posb, wnxy_ref[...], preferred_element_type=jnp.float32, precision=lax.Precision.HIGHEST)
         + jnp.dot(hb, wnhh_ref[...], preferred_element_type=jnp.float32, precision=lax.Precision.HIGHEST)
         + jnp.dot(ib, wn2_ref[...], preferred_element_type=jnp.float32, precision=lax.Precision.HIGHEST)
         + bn_ref[...])
    m = jnp.sum(u, axis=1, keepdims=True) * (1.0 / DH)
    q = jnp.sum(u * u, axis=1, keepdims=True) * (1.0 / DH)
    var = jnp.maximum(q - m * m, 0.0)
    rs = lax.rsqrt(var + EPS)
    o_ref[0] = (u - m) * rs * g2_ref[...] + b2_ref[...]


def _update(h, pos, inb, wnxy, wnhh, wn2, bn, g2, b2):
    return pl.pallas_call(
        _upd_body,
        grid=(B,),
        in_specs=[
            _bspec((1, N, DH)), _fullspec((N, DX)),
            pl.BlockSpec((2, 1, N, F), lambda b: (0, b, 0, 0)),
            _fullspec((DX, DH)), _fullspec((DH, DH)), _fullspec((F, DH)),
            _fullspec((1, DH)), _fullspec((1, DH)), _fullspec((1, DH)),
        ],
        out_specs=_bspec((1, N, DH)),
        out_shape=jax.ShapeDtypeStruct((B, N, DH), jnp.float32),
    )(h, pos, inb, wnxy, wnhh, wn2, bn, g2, b2)


def _att_body(xt_ref, h_ref, q1_ref, qb1_ref, q2_ref, qb2_ref, q3_ref, qb3_ref,
              wq_ref, bq_ref, wk_ref, bk_ref, wv_ref, bv_ref, wo_ref, bo_ref,
              d1_ref, db1_ref, d2_ref, db2_ref, d3_ref, db3_ref, o_ref):
    xtb = xt_ref[0]
    hb = h_ref[0]
    q = jnp.maximum(jnp.dot(xtb, q1_ref[...], preferred_element_type=jnp.float32, precision=lax.Precision.HIGHEST)
                    + qb1_ref[...], 0.0)
    q = jnp.maximum(jnp.dot(q, q2_ref[...], preferred_element_type=jnp.float32, precision=lax.Precision.HIGHEST)
                    + qb2_ref[...], 0.0)
    q = jnp.dot(q, q3_ref[...], preferred_element_type=jnp.float32, precision=lax.Precision.HIGHEST) + qb3_ref[...]

    qp = jnp.dot(q, wq_ref[...], preferred_element_type=jnp.float32, precision=lax.Precision.HIGHEST) + bq_ref[...]
    kp = jnp.dot(hb, wk_ref[...], preferred_element_type=jnp.float32, precision=lax.Precision.HIGHEST) + bk_ref[...]
    vp = jnp.dot(hb, wv_ref[...], preferred_element_type=jnp.float32, precision=lax.Precision.HIGHEST) + bv_ref[...]

    dh = DH // NHEAD
    scale = 1.0 / (dh ** 0.5)
    outs = []
    for hd in range(NHEAD):
        sl = slice(hd * dh, (hd + 1) * dh)
        qh = qp[:, sl]
        kh = kp[:, sl]
        vh = vp[:, sl]
        att = lax.dot_general(qh, kh, (((1,), (1,)), ((), ())),
                              preferred_element_type=jnp.float32, precision=lax.Precision.HIGHEST) * scale
        att = att - jnp.max(att, axis=1, keepdims=True)
        ex = jnp.exp(att)
        att = ex / jnp.sum(ex, axis=1, keepdims=True)
        outs.append(jnp.dot(att, vh, preferred_element_type=jnp.float32, precision=lax.Precision.HIGHEST))
    o = jnp.concatenate(outs, axis=1)
    z = q + jnp.dot(o, wo_ref[...], preferred_element_type=jnp.float32, precision=lax.Precision.HIGHEST) + bo_ref[...]
    z = jnp.maximum(jnp.dot(z, d1_ref[...], preferred_element_type=jnp.float32, precision=lax.Precision.HIGHEST)
                    + db1_ref[...], 0.0)
    z = jnp.maximum(jnp.dot(z, d2_ref[...], preferred_element_type=jnp.float32, precision=lax.Precision.HIGHEST)
                    + db2_ref[...], 0.0)
    o_ref[0] = jnp.dot(z, d3_ref[...], preferred_element_type=jnp.float32, precision=lax.Precision.HIGHEST) + db3_ref[...]


def _attdec(xt, h, qw, mha, dw):
    (q1, qb1), (q2, qb2), (q3, qb3) = qw
    (d1, db1), (d2, db2), (d3, db3) = dw
    args = (xt, h, q1, qb1, q2, qb2, q3, qb3,
            mha['Wq'], mha['bq'], mha['Wk'], mha['bk'],
            mha['Wv'], mha['bv'], mha['Wo'], mha['bo'],
            d1, db1, d2, db2, d3, db3)
    in_specs = [_bspec((1, NT, DX)), _bspec((1, N, DH))]
    for a in args[2:]:
        in_specs.append(_fullspec(a.shape))
    return pl.pallas_call(
        _att_body,
        grid=(B,),
        in_specs=in_specs,
        out_specs=_bspec((1, NT, 1)),
        out_shape=jax.ShapeDtypeStruct((B, NT, 1), jnp.float32),
    )(*args)


def kernel(xc, yc, xt, pos, senders, receivers, params):
    blk = params['blk']
    Wm, bm = blk['Wm'], blk['bm']
    Wn, bn = blk['Wn'], blk['bn']
    g1, b1 = blk['g1'], blk['b1']
    g2, b2 = blk['g2'], blk['b2']

    def padF_cols(w):
        return jnp.pad(w, ((0, 0), (0, F - DN)))

    A = Wm[:DN]
    C = Wm[DN:]
    axy = padF_cols(A[:DX])
    ahh = padF_cols(A[DX:])
    bmp = padF_cols((bm)[None, :])
    cxy = padF_cols(C[:DX])
    chh = padF_cols(C[DX:])
    g1p = padF_cols(g1[None, :])
    b1p = padF_cols(b1[None, :])

    wnxy = Wn[:DX]
    wnhh = Wn[DX:DN]
    wn2 = jnp.pad(Wn[DN:], ((0, F - DN), (0, 0)))
    bnp = bn[None, :]
    g2p = g2[None, :]
    b2p = b2[None, :]

    enc = params['enc']
    (w1, eb1), (w2, eb2), (w3, eb3) = enc
    w1x, w1y = w1[:DX], w1[DX:]
    posT = jnp.swapaxes(pos, 0, 1)

    qw = [(w, b[None, :]) for (w, b) in params['qenc']]
    dw = [(w, b[None, :]) for (w, b) in params['dec']]
    mha = {k: (v if v.ndim == 2 else v[None, :]) for k, v in params['mha'].items()}

    h = _encoder(xc, yc, posT, w1x, w1y, eb1[None, :], w2, eb2[None, :],
                 w3, eb3[None, :])

    for _ in range(STEPS):
        P, S = _proj(h, pos, axy, ahh, bmp, cxy, chh)
        G = _sc_gather(P.reshape(BN, F), S.reshape(BN, F), receivers, senders)
        M = _msgln(G, g1p, b1p)
        inb = _sc_scatter(M, receivers)
        h = _update(h, pos, inb.reshape(2, B, N, F), wnxy, wnhh, wn2,
                    bnp, g2p, b2p)

    return _attdec(xt, h, qw, mha, dw)


# speedup vs baseline: 4.7710x; 1.2744x over previous
import functools

import jax
import jax.numpy as jnp
from jax import lax
from jax.experimental import pallas as pl
from jax.experimental.pallas import tpu as pltpu
from jax.experimental.pallas import tpu_sc as plsc

B, NC, NT, N, E = 4, 1024, 1024, 2048, 16384
DX, DH, NHEAD = 2, 128, 8
DN = DH + DX
F = 144
STEPS = 3
EPS = 1e-5
BN = B * N
BE = B * E

NWORK = 32
EPW = E // NWORK
CH = 128
NCHUNK = EPW // CH


def _fullspec(shape):
    nd = len(shape)
    return pl.BlockSpec(shape, lambda b, _n=nd: (0,) * _n)


def _bspec(shape):
    nd = len(shape)
    return pl.BlockSpec(shape, lambda b, _n=nd: (b,) + (0,) * (_n - 1))


def _enc_body(xc_ref, yc_ref, posT_ref, w1x_ref, w1y_ref, b1_ref, w2_ref,
              b2_ref, w3_ref, b3_ref, h0_ref):
    xcb = xc_ref[0]
    ycb = yc_ref[0]
    posT = posT_ref[...]
    x = jnp.dot(xcb, w1x_ref[...], preferred_element_type=jnp.float32, precision=lax.Precision.HIGHEST)
    x = x + ycb * w1y_ref[...] + b1_ref[...]
    x = jnp.maximum(x, 0.0)
    x = jnp.maximum(jnp.dot(x, w2_ref[...], preferred_element_type=jnp.float32, precision=lax.Precision.HIGHEST)
                    + b2_ref[...], 0.0)
    emb = jnp.dot(x, w3_ref[...], preferred_element_type=jnp.float32, precision=lax.Precision.HIGHEST) + b3_ref[...]

    d0 = xcb[:, 0:1] - posT[0:1, :]
    d1 = xcb[:, 1:2] - posT[1:2, :]
    logits = -(d0 * d0 + d1 * d1)
    logits = logits - jnp.max(logits, axis=1, keepdims=True)
    ex = jnp.exp(logits)
    scores = ex / jnp.sum(ex, axis=1, keepdims=True)
    h0 = lax.dot_general(scores, emb, (((0,), (0,)), ((), ())),
                         preferred_element_type=jnp.float32, precision=lax.Precision.HIGHEST)
    h0_ref[0] = h0


def _encoder(xc, yc, posT, w1x, w1y, b1, w2, b2, w3, b3):
    return pl.pallas_call(
        _enc_body,
        grid=(B,),
        in_specs=[
            _bspec((1, NC, DX)), _bspec((1, NC, 1)), _fullspec((DX, N)),
            _fullspec((DX, DH)), _fullspec((1, DH)), _fullspec((1, DH)),
            _fullspec((DH, DH)), _fullspec((1, DH)),
            _fullspec((DH, DH)), _fullspec((1, DH)),
        ],
        out_specs=_bspec((1, N, DH)),
        out_shape=jax.ShapeDtypeStruct((B, N, DH), jnp.float32),
    )(xc, yc, posT, w1x, w1y, b1, w2, b2, w3, b3)


def _proj_body(h_ref, pos_ref, axy_ref, ahh_ref, bm_ref, cxy_ref, chh_ref,
               p_ref, s_ref):
    hb = h_ref[0]
    posb = pos_ref[...]
    p_ref[0] = (jnp.dot(posb, axy_ref[...], preferred_element_type=jnp.float32, precision=lax.Precision.HIGHEST)
                + jnp.dot(hb, ahh_ref[...], preferred_element_type=jnp.float32, precision=lax.Precision.HIGHEST)
                + bm_ref[...])
    s_ref[0] = (jnp.dot(posb, cxy_ref[...], preferred_element_type=jnp.float32, precision=lax.Precision.HIGHEST)
                + jnp.dot(hb, chh_ref[...], preferred_element_type=jnp.float32, precision=lax.Precision.HIGHEST))


def _proj(h, pos, axy, ahh, bm, cxy, chh):
    return pl.pallas_call(
        _proj_body,
        grid=(B,),
        in_specs=[
            _bspec((1, N, DH)), _fullspec((N, DX)),
            _fullspec((DX, F)), _fullspec((DH, F)), _fullspec((1, F)),
            _fullspec((DX, F)), _fullspec((DH, F)),
        ],
        out_specs=[_bspec((1, N, F)), _bspec((1, N, F))],
        out_shape=[jax.ShapeDtypeStruct((B, N, F), jnp.float32),
                   jax.ShapeDtypeStruct((B, N, F), jnp.float32)],
    )(h, pos, axy, ahh, bm, cxy, chh)


_LNROWS = 4096


def _msgln_body(g_ref, g1_ref, b1_ref, o_ref):
    x = g_ref[...]
    s = jnp.sum(x, axis=1, keepdims=True) * (1.0 / DN)
    q = jnp.sum(x * x, axis=1, keepdims=True) * (1.0 / DN)
    var = jnp.maximum(q - s * s, 0.0)
    rs = lax.rsqrt(var + EPS)
    o_ref[...] = (x - s) * rs * g1_ref[...] + b1_ref[...]


def _msgln(g, g1p, b1p):
    nb = BE // _LNROWS
    return pl.pallas_call(
        _msgln_body,
        grid=(nb,),
        in_specs=[_bspec((_LNROWS, F)), _fullspec((1, F)), _fullspec((1, F))],
        out_specs=_bspec((_LNROWS, F)),
        out_shape=jax.ShapeDtypeStruct((BE, F), jnp.float32),
    )(g, g1p, b1p)


def _sc_gather_body(p_hbm, s_hbm, recv_hbm, send_hbm, g_hbm,
                    idxr, idxs, prow, srow, gbuf, sem):
    cid = lax.axis_index("c")
    sid = lax.axis_index("s")
    wid = cid * 16 + sid

    def chunk(t, carry):
        b = t // (NCHUNK)
        k = t - b * NCHUNK
        off = b * N
        ebase = pl.multiple_of(wid * EPW + k * CH, CH)
        pltpu.sync_copy(recv_hbm.at[pl.ds(ebase, CH)], idxr)
        pltpu.sync_copy(send_hbm.at[pl.ds(ebase, CH)], idxs)
        for i in range(CH // 16):
            sl = pl.ds(i * 16, 16)
            idxr[sl] = idxr[sl] + off
            idxs[sl] = idxs[sl] + off
        pltpu.async_copy(p_hbm.at[idxr], prow, sem).wait()
        pltpu.async_copy(s_hbm.at[idxs], srow, sem).wait()

        def addb(e, c2):
            for j in range(F // 16):
                sl = pl.ds(j * 16, 16)
                gbuf[e, sl] = prow[e, sl] + srow[e, sl]
            return c2

        lax.fori_loop(0, CH, addb, 0)
        gb = pl.multiple_of(b * E + ebase, CH)
        pltpu.sync_copy(gbuf, g_hbm.at[pl.ds(gb, CH)])
        return carry

    lax.fori_loop(0, B * NCHUNK, chunk, 0)


def _sc_gather(p2, s2, receivers, senders):
    call = pl.kernel(
        _sc_gather_body,
        out_type=jax.ShapeDtypeStruct((BE, F), jnp.float32),
        compiler_params=pltpu.CompilerParams(use_tc_tiling_on_sc=False),
        mesh=plsc.VectorSubcoreMesh(core_axis_name="c", subcore_axis_name="s"),
        scratch_types=[
            pltpu.VMEM((CH,), jnp.int32),
            pltpu.VMEM((CH,), jnp.int32),
            pltpu.VMEM((CH, F), jnp.float32),
            pltpu.VMEM((CH, F), jnp.float32),
            pltpu.VMEM((CH, F), jnp.float32),
            pltpu.SemaphoreType.DMA,
        ],
    )
    return call(p2, s2, receivers, senders)


def _sc_scatter_body(m_hbm, recv_hbm, inb_hbm, idxr, mbuf, spmem, sem):
    cid = lax.axis_index("c")
    sid = lax.axis_index("s")

    def zb(e, c2):
        for j in range(F // 16):
            mbuf[e, pl.ds(j * 16, 16)] = jnp.zeros((16,), jnp.float32)
        return c2

    lax.fori_loop(0, CH, zb, 0)

    def zcp(k, c2):
        r0 = pl.multiple_of(sid * (BN // 16) + k * CH, CH)
        pltpu.sync_copy(mbuf, spmem.at[pl.ds(r0, CH)])
        return c2

    lax.fori_loop(0, BN // 16 // CH, zcp, 0)
    plsc.subcore_barrier()

    def sb(t, c2):
        b = t // NCHUNK
        k = t - b * NCHUNK
        ebase = pl.multiple_of(cid * (E // 2) + sid * EPW + k * CH, CH)
        pltpu.sync_copy(recv_hbm.at[pl.ds(ebase, CH)], idxr)
        for i in range(CH // 16):
            sl = pl.ds(i * 16, 16)
            idxr[sl] = idxr[sl] + b * N
        mb = pl.multiple_of(b * E + ebase, CH)
        pltpu.sync_copy(m_hbm.at[pl.ds(mb, CH)], mbuf)
        pltpu.sync_copy(mbuf, spmem.at[idxr], add=True)
        return c2

    lax.fori_loop(0, B * NCHUNK, sb, 0)
    plsc.subcore_barrier()

    def db(k, c2):
        r0 = pl.multiple_of(sid * (BN // 16) + k * CH, CH)
        pltpu.sync_copy(spmem.at[pl.ds(r0, CH)], mbuf)
        pltpu.sync_copy(mbuf, inb_hbm.at[cid, pl.ds(r0, CH)])
        return c2

    lax.fori_loop(0, BN // 16 // CH, db, 0)


def _sc_scatter(msgs, receivers):
    call = pl.kernel(
        _sc_scatter_body,
        out_type=jax.ShapeDtypeStruct((2, BN, F), jnp.float32),
        compiler_params=pltpu.CompilerParams(use_tc_tiling_on_sc=False),
        mesh=plsc.VectorSubcoreMesh(core_axis_name="c", subcore_axis_name="s"),
        scratch_types=[
            pltpu.VMEM((CH,), jnp.int32),
            pltpu.VMEM((CH, F), jnp.float32),
            pltpu.VMEM_SHARED((BN, F), jnp.float32),
            pltpu.SemaphoreType.DMA,
        ],
    )
    return call(msgs, receivers)


def _upd_body(h_ref, pos_ref, inb_ref, wnxy_ref, wnhh_ref, wn2_ref, bn_ref,
              g2_ref, b2_ref, o_ref):
    hb = h_ref[0]
    posb = pos_ref[...]
    ib = inb_ref[0, 0] + inb_ref[1, 0]
    u = (jnp.dot(posb, wnxy_ref[...], preferred_element_type=jnp.float32, precision=lax.Precision.HIGHEST)
         + jnp.dot(hb, wnhh_ref[...], preferred_element_type=jnp.float32, precision=lax.Precision.HIGHEST)
         + jnp.dot(ib, wn2_ref[...], preferred_element_type=jnp.float32, precision=lax.Precision.HIGHEST)
         + bn_ref[...])
    m = jnp.sum(u, axis=1, keepdims=True) * (1.0 / DH)
    q = jnp.sum(u * u, axis=1, keepdims=True) * (1.0 / DH)
    var = jnp.maximum(q - m * m, 0.0)
    rs = lax.rsqrt(var + EPS)
    o_ref[0] = (u - m) * rs * g2_ref[...] + b2_ref[...]


def _update(h, pos, inb, wnxy, wnhh, wn2, bn, g2, b2):
    return pl.pallas_call(
        _upd_body,
        grid=(B,),
        in_specs=[
            _bspec((1, N, DH)), _fullspec((N, DX)),
            pl.BlockSpec((2, 1, N, F), lambda b: (0, b, 0, 0)),
            _fullspec((DX, DH)), _fullspec((DH, DH)), _fullspec((F, DH)),
            _fullspec((1, DH)), _fullspec((1, DH)), _fullspec((1, DH)),
        ],
        out_specs=_bspec((1, N, DH)),
        out_shape=jax.ShapeDtypeStruct((B, N, DH), jnp.float32),
    )(h, pos, inb, wnxy, wnhh, wn2, bn, g2, b2)


def _att_body(xt_ref, h_ref, q1_ref, qb1_ref, q2_ref, qb2_ref, q3_ref, qb3_ref,
              wq_ref, bq_ref, wk_ref, bk_ref, wv_ref, bv_ref, wo_ref, bo_ref,
              d1_ref, db1_ref, d2_ref, db2_ref, d3_ref, db3_ref, o_ref):
    xtb = xt_ref[0]
    hb = h_ref[0]
    q = jnp.maximum(jnp.dot(xtb, q1_ref[...], preferred_element_type=jnp.float32, precision=lax.Precision.HIGHEST)
                    + qb1_ref[...], 0.0)
    q = jnp.maximum(jnp.dot(q, q2_ref[...], preferred_element_type=jnp.float32, precision=lax.Precision.HIGHEST)
                    + qb2_ref[...], 0.0)
    q = jnp.dot(q, q3_ref[...], preferred_element_type=jnp.float32, precision=lax.Precision.HIGHEST) + qb3_ref[...]

    qp = jnp.dot(q, wq_ref[...], preferred_element_type=jnp.float32, precision=lax.Precision.HIGHEST) + bq_ref[...]
    kp = jnp.dot(hb, wk_ref[...], preferred_element_type=jnp.float32, precision=lax.Precision.HIGHEST) + bk_ref[...]
    vp = jnp.dot(hb, wv_ref[...], preferred_element_type=jnp.float32, precision=lax.Precision.HIGHEST) + bv_ref[...]

    dh = DH // NHEAD
    scale = 1.0 / (dh ** 0.5)
    outs = []
    for hd in range(NHEAD):
        sl = slice(hd * dh, (hd + 1) * dh)
        qh = qp[:, sl]
        kh = kp[:, sl]
        vh = vp[:, sl]
        att = lax.dot_general(qh, kh, (((1,), (1,)), ((), ())),
                              preferred_element_type=jnp.float32) * scale
        att = att - jnp.max(att, axis=1, keepdims=True)
        ex = jnp.exp(att)
        att = ex / jnp.sum(ex, axis=1, keepdims=True)
        outs.append(jnp.dot(att, vh, preferred_element_type=jnp.float32))
    o = jnp.concatenate(outs, axis=1)
    z = q + jnp.dot(o, wo_ref[...], preferred_element_type=jnp.float32, precision=lax.Precision.HIGHEST) + bo_ref[...]
    z = jnp.maximum(jnp.dot(z, d1_ref[...], preferred_element_type=jnp.float32, precision=lax.Precision.HIGHEST)
                    + db1_ref[...], 0.0)
    z = jnp.maximum(jnp.dot(z, d2_ref[...], preferred_element_type=jnp.float32, precision=lax.Precision.HIGHEST)
                    + db2_ref[...], 0.0)
    o_ref[0] = jnp.dot(z, d3_ref[...], preferred_element_type=jnp.float32, precision=lax.Precision.HIGHEST) + db3_ref[...]


def _attdec(xt, h, qw, mha, dw):
    (q1, qb1), (q2, qb2), (q3, qb3) = qw
    (d1, db1), (d2, db2), (d3, db3) = dw
    args = (xt, h, q1, qb1, q2, qb2, q3, qb3,
            mha['Wq'], mha['bq'], mha['Wk'], mha['bk'],
            mha['Wv'], mha['bv'], mha['Wo'], mha['bo'],
            d1, db1, d2, db2, d3, db3)
    in_specs = [_bspec((1, NT, DX)), _bspec((1, N, DH))]
    for a in args[2:]:
        in_specs.append(_fullspec(a.shape))
    return pl.pallas_call(
        _att_body,
        grid=(B,),
        in_specs=in_specs,
        out_specs=_bspec((1, NT, 1)),
        out_shape=jax.ShapeDtypeStruct((B, NT, 1), jnp.float32),
    )(*args)


def kernel(xc, yc, xt, pos, senders, receivers, params):
    blk = params['blk']
    Wm, bm = blk['Wm'], blk['bm']
    Wn, bn = blk['Wn'], blk['bn']
    g1, b1 = blk['g1'], blk['b1']
    g2, b2 = blk['g2'], blk['b2']

    def padF_cols(w):
        return jnp.pad(w, ((0, 0), (0, F - DN)))

    A = Wm[:DN]
    C = Wm[DN:]
    axy = padF_cols(A[:DX])
    ahh = padF_cols(A[DX:])
    bmp = padF_cols((bm)[None, :])
    cxy = padF_cols(C[:DX])
    chh = padF_cols(C[DX:])
    g1p = padF_cols(g1[None, :])
    b1p = padF_cols(b1[None, :])

    wnxy = Wn[:DX]
    wnhh = Wn[DX:DN]
    wn2 = jnp.pad(Wn[DN:], ((0, F - DN), (0, 0)))
    bnp = bn[None, :]
    g2p = g2[None, :]
    b2p = b2[None, :]

    enc = params['enc']
    (w1, eb1), (w2, eb2), (w3, eb3) = enc
    w1x, w1y = w1[:DX], w1[DX:]
    posT = jnp.swapaxes(pos, 0, 1)

    qw = [(w, b[None, :]) for (w, b) in params['qenc']]
    dw = [(w, b[None, :]) for (w, b) in params['dec']]
    mha = {k: (v if v.ndim == 2 else v[None, :]) for k, v in params['mha'].items()}

    h = _encoder(xc, yc, posT, w1x, w1y, eb1[None, :], w2, eb2[None, :],
                 w3, eb3[None, :])

    for _ in range(STEPS):
        P, S = _proj(h, pos, axy, ahh, bmp, cxy, chh)
        G = _sc_gather(P.reshape(BN, F), S.reshape(BN, F), receivers, senders)
        M = _msgln(G, g1p, b1p)
        inb = _sc_scatter(M, receivers)
        h = _update(h, pos, inb.reshape(2, B, N, F), wnxy, wnhh, wn2,
                    bnp, g2p, b2p)

    return _attdec(xt, h, qw, mha, dw)


# speedup vs baseline: 7.4287x; 1.5570x over previous
import functools

import jax
import jax.numpy as jnp
from jax import lax
from jax.experimental import pallas as pl
from jax.experimental.pallas import tpu as pltpu
from jax.experimental.pallas import tpu_sc as plsc

B, NC, NT, N, E = 4, 1024, 1024, 2048, 16384
DX, DH, NHEAD = 2, 128, 8
DN = DH + DX
F = 144
STEPS = 3
EPS = 1e-5
BN = B * N
BE = B * E

NWORK = 32
EPW = E // NWORK
CH = 128
NCHUNK = EPW // CH


def _fullspec(shape):
    nd = len(shape)
    return pl.BlockSpec(shape, lambda b, _n=nd: (0,) * _n)


def _bspec(shape):
    nd = len(shape)
    return pl.BlockSpec(shape, lambda b, _n=nd: (b,) + (0,) * (_n - 1))


def _enc_body(xc_ref, yc_ref, posT_ref, w1x_ref, w1y_ref, b1_ref, w2_ref,
              b2_ref, w3_ref, b3_ref, h0_ref):
    xcb = xc_ref[0]
    ycb = yc_ref[0]
    posT = posT_ref[...]
    x = jnp.dot(xcb, w1x_ref[...], preferred_element_type=jnp.float32, precision=lax.Precision.HIGHEST)
    x = x + ycb * w1y_ref[...] + b1_ref[...]
    x = jnp.maximum(x, 0.0)
    x = jnp.maximum(jnp.dot(x, w2_ref[...], preferred_element_type=jnp.float32, precision=lax.Precision.HIGHEST)
                    + b2_ref[...], 0.0)
    emb = jnp.dot(x, w3_ref[...], preferred_element_type=jnp.float32, precision=lax.Precision.HIGHEST) + b3_ref[...]

    d0 = xcb[:, 0:1] - posT[0:1, :]
    d1 = xcb[:, 1:2] - posT[1:2, :]
    logits = -(d0 * d0 + d1 * d1)
    logits = logits - jnp.max(logits, axis=1, keepdims=True)
    ex = jnp.exp(logits)
    scores = ex / jnp.sum(ex, axis=1, keepdims=True)
    h0 = lax.dot_general(scores, emb, (((0,), (0,)), ((), ())),
                         preferred_element_type=jnp.float32, precision=lax.Precision.HIGHEST)
    h0_ref[0] = h0


def _encoder(xc, yc, posT, w1x, w1y, b1, w2, b2, w3, b3):
    return pl.pallas_call(
        _enc_body,
        grid=(B,),
        in_specs=[
            _bspec((1, NC, DX)), _bspec((1, NC, 1)), _fullspec((DX, N)),
            _fullspec((DX, DH)), _fullspec((1, DH)), _fullspec((1, DH)),
            _fullspec((DH, DH)), _fullspec((1, DH)),
            _fullspec((DH, DH)), _fullspec((1, DH)),
        ],
        out_specs=_bspec((1, N, DH)),
        out_shape=jax.ShapeDtypeStruct((B, N, DH), jnp.float32),
    )(xc, yc, posT, w1x, w1y, b1, w2, b2, w3, b3)


def _proj_body(h_ref, pos_ref, axy_ref, ahh_ref, bm_ref, cxy_ref, chh_ref,
               p_ref, s_ref):
    hb = h_ref[0]
    posb = pos_ref[...]
    p_ref[0] = (jnp.dot(posb, axy_ref[...], preferred_element_type=jnp.float32, precision=lax.Precision.HIGHEST)
                + jnp.dot(hb, ahh_ref[...], preferred_element_type=jnp.float32, precision=lax.Precision.HIGHEST)
                + bm_ref[...])
    s_ref[0] = (jnp.dot(posb, cxy_ref[...], preferred_element_type=jnp.float32, precision=lax.Precision.HIGHEST)
                + jnp.dot(hb, chh_ref[...], preferred_element_type=jnp.float32, precision=lax.Precision.HIGHEST))


def _proj(h, pos, axy, ahh, bm, cxy, chh):
    return pl.pallas_call(
        _proj_body,
        grid=(B,),
        in_specs=[
            _bspec((1, N, DH)), _fullspec((N, DX)),
            _fullspec((DX, F)), _fullspec((DH, F)), _fullspec((1, F)),
            _fullspec((DX, F)), _fullspec((DH, F)),
        ],
        out_specs=[_bspec((1, N, F)), _bspec((1, N, F))],
        out_shape=[jax.ShapeDtypeStruct((B, N, F), jnp.float32),
                   jax.ShapeDtypeStruct((B, N, F), jnp.float32)],
    )(h, pos, axy, ahh, bm, cxy, chh)


_LNROWS = 4096


def _msgln_body(g_ref, g1_ref, b1_ref, o_ref):
    x = g_ref[...]
    s = jnp.sum(x, axis=1, keepdims=True) * (1.0 / DN)
    q = jnp.sum(x * x, axis=1, keepdims=True) * (1.0 / DN)
    var = jnp.maximum(q - s * s, 0.0)
    rs = lax.rsqrt(var + EPS)
    o_ref[...] = (x - s) * rs * g1_ref[...] + b1_ref[...]


def _msgln(g, g1p, b1p):
    nb = BE // _LNROWS
    return pl.pallas_call(
        _msgln_body,
        grid=(nb,),
        in_specs=[_bspec((_LNROWS, F)), _fullspec((1, F)), _fullspec((1, F))],
        out_specs=_bspec((_LNROWS, F)),
        out_shape=jax.ShapeDtypeStruct((BE, F), jnp.float32),
    )(g, g1p, b1p)


def _sc_gather_body(p_hbm, s_hbm, recv_hbm, send_hbm, g_hbm,
                    idxr, idxs, prow, srow, gbuf, sem):
    cid = lax.axis_index("c")
    sid = lax.axis_index("s")
    wid = cid * 16 + sid

    def chunk(t, carry):
        b = t // (NCHUNK)
        k = t - b * NCHUNK
        off = b * N
        ebase = pl.multiple_of(wid * EPW + k * CH, CH)
        pltpu.sync_copy(recv_hbm.at[pl.ds(ebase, CH)], idxr)
        pltpu.sync_copy(send_hbm.at[pl.ds(ebase, CH)], idxs)
        for i in range(CH // 16):
            sl = pl.ds(i * 16, 16)
            idxr[sl] = idxr[sl] + off
            idxs[sl] = idxs[sl] + off
        pltpu.async_copy(p_hbm.at[idxr], prow, sem).wait()
        pltpu.async_copy(s_hbm.at[idxs], srow, sem).wait()

        def addb(e, c2):
            for j in range(F // 16):
                sl = pl.ds(j * 16, 16)
                gbuf[e, sl] = prow[e, sl] + srow[e, sl]
            return c2

        lax.fori_loop(0, CH, addb, 0)
        gb = pl.multiple_of(b * E + ebase, CH)
        pltpu.sync_copy(gbuf, g_hbm.at[pl.ds(gb, CH)])
        return carry

    lax.fori_loop(0, B * NCHUNK, chunk, 0)


def _sc_gather(p2, s2, receivers, senders):
    call = pl.kernel(
        _sc_gather_body,
        out_type=jax.ShapeDtypeStruct((BE, F), jnp.float32),
        compiler_params=pltpu.CompilerParams(use_tc_tiling_on_sc=False),
        mesh=plsc.VectorSubcoreMesh(core_axis_name="c", subcore_axis_name="s"),
        scratch_types=[
            pltpu.VMEM((CH,), jnp.int32),
            pltpu.VMEM((CH,), jnp.int32),
            pltpu.VMEM((CH, F), jnp.float32),
            pltpu.VMEM((CH, F), jnp.float32),
            pltpu.VMEM((CH, F), jnp.float32),
            pltpu.SemaphoreType.DMA,
        ],
    )
    return call(p2, s2, receivers, senders)


def _sc_scatter_body(m_hbm, recv_hbm, inb_hbm, idxr, mbuf, spmem, sem):
    cid = lax.axis_index("c")
    sid = lax.axis_index("s")

    def zb(e, c2):
        for j in range(F // 16):
            mbuf[e, pl.ds(j * 16, 16)] = jnp.zeros((16,), jnp.float32)
        return c2

    lax.fori_loop(0, CH, zb, 0)

    def zcp(k, c2):
        r0 = pl.multiple_of(sid * (BN // 16) + k * CH, CH)
        pltpu.sync_copy(mbuf, spmem.at[pl.ds(r0, CH)])
        return c2

    lax.fori_loop(0, BN // 16 // CH, zcp, 0)
    plsc.subcore_barrier()

    def sb(t, c2):
        b = t // NCHUNK
        k = t - b * NCHUNK
        ebase = pl.multiple_of(cid * (E // 2) + sid * EPW + k * CH, CH)
        pltpu.sync_copy(recv_hbm.at[pl.ds(ebase, CH)], idxr)
        for i in range(CH // 16):
            sl = pl.ds(i * 16, 16)
            idxr[sl] = idxr[sl] + b * N
        mb = pl.multiple_of(b * E + ebase, CH)
        pltpu.sync_copy(m_hbm.at[pl.ds(mb, CH)], mbuf)
        pltpu.sync_copy(mbuf, spmem.at[idxr], add=True)
        return c2

    lax.fori_loop(0, B * NCHUNK, sb, 0)
    plsc.subcore_barrier()

    def db(k, c2):
        r0 = pl.multiple_of(sid * (BN // 16) + k * CH, CH)
        pltpu.sync_copy(spmem.at[pl.ds(r0, CH)], mbuf)
        pltpu.sync_copy(mbuf, inb_hbm.at[cid, pl.ds(r0, CH)])
        return c2

    lax.fori_loop(0, BN // 16 // CH, db, 0)


def _sc_scatter(msgs, receivers):
    call = pl.kernel(
        _sc_scatter_body,
        out_type=jax.ShapeDtypeStruct((2, BN, F), jnp.float32),
        compiler_params=pltpu.CompilerParams(use_tc_tiling_on_sc=False),
        mesh=plsc.VectorSubcoreMesh(core_axis_name="c", subcore_axis_name="s"),
        scratch_types=[
            pltpu.VMEM((CH,), jnp.int32),
            pltpu.VMEM((CH, F), jnp.float32),
            pltpu.VMEM_SHARED((BN, F), jnp.float32),
            pltpu.SemaphoreType.DMA,
        ],
    )
    return call(msgs, receivers)


_NEWTON = 3


def _sc_edge_body(p_hbm, s_hbm, recv_hbm, send_hbm, g1_hbm, b1_hbm, inb_hbm,
                  idxr, idxs, prow, srow, obuf, g1v, b1v, spmem, sem):
    cid = lax.axis_index("c")
    sid = lax.axis_index("s")
    nj = F // 16

    pltpu.sync_copy(g1_hbm, g1v)
    pltpu.sync_copy(b1_hbm, b1v)
    g1s = [g1v[pl.ds(j * 16, 16)] for j in range(nj)]
    b1s = [b1v[pl.ds(j * 16, 16)] for j in range(nj)]

    def zb(e, c2):
        for j in range(nj):
            obuf[e, pl.ds(j * 16, 16)] = jnp.zeros((16,), jnp.float32)
        return c2

    lax.fori_loop(0, CH, zb, 0)

    def zcp(k, c2):
        r0 = pl.multiple_of(sid * (BN // 16) + k * CH, CH)
        pltpu.sync_copy(obuf, spmem.at[pl.ds(r0, CH)])
        return c2

    lax.fori_loop(0, BN // 16 // CH, zcp, 0)
    plsc.subcore_barrier()

    def chunk(t, carry):
        b = t // NCHUNK
        k = t - b * NCHUNK
        off = b * N
        ebase = pl.multiple_of(cid * (E // 2) + sid * EPW + k * CH, CH)
        pltpu.sync_copy(recv_hbm.at[pl.ds(ebase, CH)], idxr)
        pltpu.sync_copy(send_hbm.at[pl.ds(ebase, CH)], idxs)
        for i in range(CH // 16):
            sl = pl.ds(i * 16, 16)
            idxr[sl] = idxr[sl] + off
            idxs[sl] = idxs[sl] + off
        pltpu.async_copy(p_hbm.at[idxr], prow, sem).wait()
        pltpu.async_copy(s_hbm.at[idxs], srow, sem).wait()

        def ln_edge(e, c2):
            vs = [prow[e, pl.ds(j * 16, 16)] + srow[e, pl.ds(j * 16, 16)]
                  for j in range(nj)]
            t1 = vs[0]
            for j in range(1, nj):
                t1 = t1 + vs[j]
            m = jnp.sum(t1) * (1.0 / DN)
            ds = [vs[j] - m for j in range(nj)]
            sq = ds[0] * ds[0]
            for j in range(1, nj):
                sq = sq + ds[j] * ds[j]
            qsum = jnp.sum(sq) - (F - DN) * m * m
            var = jnp.maximum(qsum * (1.0 / DN), 0.0) + EPS
            xb = lax.bitcast_convert_type(var, jnp.int32)
            yb = jnp.int32(0x5F3759DF) - lax.shift_right_logical(xb, 1)
            y = lax.bitcast_convert_type(yb, jnp.float32)
            vh = 0.5 * var
            for _ in range(_NEWTON):
                y = y * (1.5 - vh * y * y)
            for j in range(nj):
                obuf[e, pl.ds(j * 16, 16)] = ds[j] * y * g1s[j] + b1s[j]
            return c2

        lax.fori_loop(0, CH, ln_edge, 0)
        pltpu.sync_copy(obuf, spmem.at[idxr], add=True)
        return carry

    lax.fori_loop(0, B * NCHUNK, chunk, 0)
    plsc.subcore_barrier()

    def db(k, c2):
        r0 = pl.multiple_of(sid * (BN // 16) + k * CH, CH)
        pltpu.sync_copy(spmem.at[pl.ds(r0, CH)], prow)
        pltpu.sync_copy(prow, inb_hbm.at[cid, pl.ds(r0, CH)])
        return c2

    lax.fori_loop(0, BN // 16 // CH, db, 0)


def _sc_edge(p2, s2, receivers, senders, g1f, b1f):
    call = pl.kernel(
        _sc_edge_body,
        out_type=jax.ShapeDtypeStruct((2, BN, F), jnp.float32),
        compiler_params=pltpu.CompilerParams(use_tc_tiling_on_sc=False,
                                             needs_layout_passes=False),
        mesh=plsc.VectorSubcoreMesh(core_axis_name="c", subcore_axis_name="s"),
        scratch_types=[
            pltpu.VMEM((CH,), jnp.int32),
            pltpu.VMEM((CH,), jnp.int32),
            pltpu.VMEM((CH, F), jnp.float32),
            pltpu.VMEM((CH, F), jnp.float32),
            pltpu.VMEM((CH, F), jnp.float32),
            pltpu.VMEM((F,), jnp.float32),
            pltpu.VMEM((F,), jnp.float32),
            pltpu.VMEM_SHARED((BN, F), jnp.float32),
            pltpu.SemaphoreType.DMA,
        ],
    )
    return call(p2, s2, receivers, senders, g1f, b1f)


def _upd_body(h_ref, pos_ref, inb_ref, wnxy_ref, wnhh_ref, wn2_ref, bn_ref,
              g2_ref, b2_ref, o_ref):
    hb = h_ref[0]
    posb = pos_ref[...]
    ib = inb_ref[0, 0] + inb_ref[1, 0]
    u = (jnp.dot(posb, wnxy_ref[...], preferred_element_type=jnp.float32, precision=lax.Precision.HIGHEST)
         + jnp.dot(hb, wnhh_ref[...], preferred_element_type=jnp.float32, precision=lax.Precision.HIGHEST)
         + jnp.dot(ib, wn2_ref[...], preferred_element_type=jnp.float32, precision=lax.Precision.HIGHEST)
         + bn_ref[...])
    m = jnp.sum(u, axis=1, keepdims=True) * (1.0 / DH)
    d = u - m
    var = jnp.sum(d * d, axis=1, keepdims=True) * (1.0 / DH)
    rs = lax.rsqrt(var + EPS)
    o_ref[0] = d * rs * g2_ref[...] + b2_ref[...]


def _update(h, pos, inb, wnxy, wnhh, wn2, bn, g2, b2):
    return pl.pallas_call(
        _upd_body,
        grid=(B,),
        in_specs=[
            _bspec((1, N, DH)), _fullspec((N, DX)),
            pl.BlockSpec((2, 1, N, F), lambda b: (0, b, 0, 0)),
            _fullspec((DX, DH)), _fullspec((DH, DH)), _fullspec((F, DH)),
            _fullspec((1, DH)), _fullspec((1, DH)), _fullspec((1, DH)),
        ],
        out_specs=_bspec((1, N, DH)),
        out_shape=jax.ShapeDtypeStruct((B, N, DH), jnp.float32),
    )(h, pos, inb, wnxy, wnhh, wn2, bn, g2, b2)


def _updproj_body(h_ref, pos_ref, inb_ref, wnxy_ref, wnhh_ref, wn2_ref, bn_ref,
                  g2_ref, b2_ref, axy_ref, ahh_ref, bm_ref, cxy_ref, chh_ref,
                  o_ref, p_ref, s_ref):
    hb = h_ref[0]
    posb = pos_ref[...]
    ib = inb_ref[0, 0] + inb_ref[1, 0]
    u = (jnp.dot(posb, wnxy_ref[...], preferred_element_type=jnp.float32, precision=lax.Precision.HIGHEST)
         + jnp.dot(hb, wnhh_ref[...], preferred_element_type=jnp.float32, precision=lax.Precision.HIGHEST)
         + jnp.dot(ib, wn2_ref[...], preferred_element_type=jnp.float32, precision=lax.Precision.HIGHEST)
         + bn_ref[...])
    m = jnp.sum(u, axis=1, keepdims=True) * (1.0 / DH)
    d = u - m
    var = jnp.sum(d * d, axis=1, keepdims=True) * (1.0 / DH)
    rs = lax.rsqrt(var + EPS)
    hn = d * rs * g2_ref[...] + b2_ref[...]
    o_ref[0] = hn
    p_ref[0] = (jnp.dot(posb, axy_ref[...], preferred_element_type=jnp.float32, precision=lax.Precision.HIGHEST)
                + jnp.dot(hn, ahh_ref[...], preferred_element_type=jnp.float32, precision=lax.Precision.HIGHEST)
                + bm_ref[...])
    s_ref[0] = (jnp.dot(posb, cxy_ref[...], preferred_element_type=jnp.float32, precision=lax.Precision.HIGHEST)
                + jnp.dot(hn, chh_ref[...], preferred_element_type=jnp.float32, precision=lax.Precision.HIGHEST))


def _update_proj(h, pos, inb, wnxy, wnhh, wn2, bn, g2, b2, axy, ahh, bm, cxy, chh):
    return pl.pallas_call(
        _updproj_body,
        grid=(B,),
        in_specs=[
            _bspec((1, N, DH)), _fullspec((N, DX)),
            pl.BlockSpec((2, 1, N, F), lambda b: (0, b, 0, 0)),
            _fullspec((DX, DH)), _fullspec((DH, DH)), _fullspec((F, DH)),
            _fullspec((1, DH)), _fullspec((1, DH)), _fullspec((1, DH)),
            _fullspec((DX, F)), _fullspec((DH, F)), _fullspec((1, F)),
            _fullspec((DX, F)), _fullspec((DH, F)),
        ],
        out_specs=[_bspec((1, N, DH)), _bspec((1, N, F)), _bspec((1, N, F))],
        out_shape=[jax.ShapeDtypeStruct((B, N, DH), jnp.float32),
                   jax.ShapeDtypeStruct((B, N, F), jnp.float32),
                   jax.ShapeDtypeStruct((B, N, F), jnp.float32)],
    )(h, pos, inb, wnxy, wnhh, wn2, bn, g2, b2, axy, ahh, bm, cxy, chh)


def _att_body(xt_ref, h_ref, q1_ref, qb1_ref, q2_ref, qb2_ref, q3_ref, qb3_ref,
              wq_ref, bq_ref, wk_ref, bk_ref, wv_ref, bv_ref, wo_ref, bo_ref,
              d1_ref, db1_ref, d2_ref, db2_ref, d3_ref, db3_ref, o_ref):
    xtb = xt_ref[0]
    hb = h_ref[0]
    q = jnp.maximum(jnp.dot(xtb, q1_ref[...], preferred_element_type=jnp.float32, precision=lax.Precision.HIGHEST)
                    + qb1_ref[...], 0.0)
    q = jnp.maximum(jnp.dot(q, q2_ref[...], preferred_element_type=jnp.float32, precision=lax.Precision.HIGHEST)
                    + qb2_ref[...], 0.0)
    q = jnp.dot(q, q3_ref[...], preferred_element_type=jnp.float32, precision=lax.Precision.HIGHEST) + qb3_ref[...]

    qp = jnp.dot(q, wq_ref[...], preferred_element_type=jnp.float32, precision=lax.Precision.HIGHEST) + bq_ref[...]
    kp = jnp.dot(hb, wk_ref[...], preferred_element_type=jnp.float32, precision=lax.Precision.HIGHEST) + bk_ref[...]
    vp = jnp.dot(hb, wv_ref[...], preferred_element_type=jnp.float32, precision=lax.Precision.HIGHEST) + bv_ref[...]

    dh = DH // NHEAD
    scale = 1.0 / (dh ** 0.5)
    outs = []
    for hd in range(NHEAD):
        sl = slice(hd * dh, (hd + 1) * dh)
        qh = qp[:, sl]
        kh = kp[:, sl]
        vh = vp[:, sl]
        att = lax.dot_general(qh, kh, (((1,), (1,)), ((), ())),
                              preferred_element_type=jnp.float32) * scale
        att = att - jnp.max(att, axis=1, keepdims=True)
        ex = jnp.exp(att)
        att = ex / jnp.sum(ex, axis=1, keepdims=True)
        outs.append(jnp.dot(att, vh, preferred_element_type=jnp.float32))
    o = jnp.concatenate(outs, axis=1)
    z = q + jnp.dot(o, wo_ref[...], preferred_element_type=jnp.float32, precision=lax.Precision.HIGHEST) + bo_ref[...]
    z = jnp.maximum(jnp.dot(z, d1_ref[...], preferred_element_type=jnp.float32, precision=lax.Precision.HIGHEST)
                    + db1_ref[...], 0.0)
    z = jnp.maximum(jnp.dot(z, d2_ref[...], preferred_element_type=jnp.float32, precision=lax.Precision.HIGHEST)
                    + db2_ref[...], 0.0)
    o_ref[0] = jnp.dot(z, d3_ref[...], preferred_element_type=jnp.float32, precision=lax.Precision.HIGHEST) + db3_ref[...]


def _attdec(xt, h, qw, mha, dw):
    (q1, qb1), (q2, qb2), (q3, qb3) = qw
    (d1, db1), (d2, db2), (d3, db3) = dw
    args = (xt, h, q1, qb1, q2, qb2, q3, qb3,
            mha['Wq'], mha['bq'], mha['Wk'], mha['bk'],
            mha['Wv'], mha['bv'], mha['Wo'], mha['bo'],
            d1, db1, d2, db2, d3, db3)
    in_specs = [_bspec((1, NT, DX)), _bspec((1, N, DH))]
    for a in args[2:]:
        in_specs.append(_fullspec(a.shape))
    return pl.pallas_call(
        _att_body,
        grid=(B,),
        in_specs=in_specs,
        out_specs=_bspec((1, NT, 1)),
        out_shape=jax.ShapeDtypeStruct((B, NT, 1), jnp.float32),
    )(*args)


def kernel(xc, yc, xt, pos, senders, receivers, params):
    blk = params['blk']
    Wm, bm = blk['Wm'], blk['bm']
    Wn, bn = blk['Wn'], blk['bn']
    g1, b1 = blk['g1'], blk['b1']
    g2, b2 = blk['g2'], blk['b2']

    def padF_cols(w):
        return jnp.pad(w, ((0, 0), (0, F - DN)))

    A = Wm[:DN]
    C = Wm[DN:]
    axy = padF_cols(A[:DX])
    ahh = padF_cols(A[DX:])
    bmp = padF_cols((bm)[None, :])
    cxy = padF_cols(C[:DX])
    chh = padF_cols(C[DX:])
    g1p = padF_cols(g1[None, :])
    b1p = padF_cols(b1[None, :])

    wnxy = Wn[:DX]
    wnhh = Wn[DX:DN]
    wn2 = jnp.pad(Wn[DN:], ((0, F - DN), (0, 0)))
    bnp = bn[None, :]
    g2p = g2[None, :]
    b2p = b2[None, :]

    enc = params['enc']
    (w1, eb1), (w2, eb2), (w3, eb3) = enc
    w1x, w1y = w1[:DX], w1[DX:]
    posT = jnp.swapaxes(pos, 0, 1)

    qw = [(w, b[None, :]) for (w, b) in params['qenc']]
    dw = [(w, b[None, :]) for (w, b) in params['dec']]
    mha = {k: (v if v.ndim == 2 else v[None, :]) for k, v in params['mha'].items()}

    g1f = jnp.pad(g1, (0, F - DN))
    b1f = jnp.pad(b1, (0, F - DN))

    h = _encoder(xc, yc, posT, w1x, w1y, eb1[None, :], w2, eb2[None, :],
                 w3, eb3[None, :])
    P, S = _proj(h, pos, axy, ahh, bmp, cxy, chh)

    for step in range(STEPS):
        inb = _sc_edge(P.reshape(BN, F), S.reshape(BN, F), receivers, senders,
                       g1f, b1f)
        if step < STEPS - 1:
            h, P, S = _update_proj(h, pos, inb.reshape(2, B, N, F), wnxy, wnhh,
                                   wn2, bnp, g2p, b2p, axy, ahh, bmp, cxy, chh)
        else:
            h = _update(h, pos, inb.reshape(2, B, N, F), wnxy, wnhh, wn2,
                        bnp, g2p, b2p)

    return _attdec(xt, h, qw, mha, dw)
